# baseline (speedup 1.0000x reference)
import functools

import jax
import jax.numpy as jnp
import numpy as np
from jax.experimental import pallas as pl
from jax.experimental.pallas import tpu as pltpu


def _prep_kernel(e_ref, eb_ref, u_ref, *, scale):
    e = e_ref[...] * scale
    eb_ref[...] = e.astype(jnp.bfloat16)
    u_ref[...] = 0.5 * jnp.sum(e * e, axis=1, keepdims=True)


def _loss_kernel(m_ref, ebt_ref, rm0_ref, rm1_ref, cm_ref,
                 out_ref, ga_ref, gb_ref, *, bm, bn):
    s = pl.program_id(0)
    k0 = 2 * s

    @pl.when(s == 0)
    def _init():
        ga_ref[...] = jnp.zeros_like(ga_ref)
        gb_ref[...] = jnp.zeros_like(gb_ref)
        out_ref[...] = jnp.zeros_like(out_ref)

    def epilogue(g_ref, rm_ref, k):
        bj = m_ref[1, k]
        w = m_ref[2, k].astype(jnp.float32)
        col = pl.multiple_of(bj * bn, bn)
        li = rm_ref[:, 0:1]
        ui = rm_ref[:, 1:2]
        lj = cm_ref[0:1, pl.ds(col, bn)]
        uj = cm_ref[1:2, pl.ds(col, bn)]
        g = g_ref[...]
        mse = (ui + uj) - g
        val = jnp.where(li == lj, mse, jnp.maximum(1.0 - mse, 0.0))
        part = jnp.sum(val.reshape(bm // 8, 8, bn), axis=0)
        out_ref[...] += part * w

    def matmul(g_ref, k):
        bi = m_ref[0, k]
        bj = m_ref[1, k]
        row = pl.multiple_of(bi * bm, bm)
        col = pl.multiple_of(bj * bn, bn)
        ei = ebt_ref[:, pl.ds(row, bm)]
        ejt = ebt_ref[:, pl.ds(col, bn)]
        g_ref[...] = jax.lax.dot_general(
            ei, ejt, (((0,), (0,)), ((), ())),
            preferred_element_type=jnp.float32)

    epilogue(ga_ref, rm0_ref, k0)
    epilogue(gb_ref, rm1_ref, k0 + 1)
    matmul(ga_ref, k0 + 2)
    matmul(gb_ref, k0 + 3)


def kernel(embeddings, labels):
    B, D = embeddings.shape
    BM = 512
    BN = 512
    nb = B // BM

    labf = labels.astype(jnp.float32)
    scale = 1.0 / float(np.sqrt(D / 2))

    NB = 1024
    eb, u_col = pl.pallas_call(
        functools.partial(_prep_kernel, scale=scale),
        grid=(B // NB,),
        in_specs=[pl.BlockSpec((NB, D), lambda i: (i, 0))],
        out_specs=[
            pl.BlockSpec((NB, D), lambda i: (i, 0)),
            pl.BlockSpec((NB, 1), lambda i: (i, 0)),
        ],
        out_shape=[
            jax.ShapeDtypeStruct((B, D), jnp.bfloat16),
            jax.ShapeDtypeStruct((B, 1), jnp.float32),
        ],
    )(embeddings)

    ebt = eb.T
    u = u_col[:, 0]
    colmeta = jnp.stack([labf, u], axis=0)
    rowmeta = jnp.stack([labf, u], axis=1)

    pairs = [(i, j, 1 if i == j else 2)
             for i in range(nb) for j in range(i, nb)]
    L = len(pairs)
    slots = [(0, 0, 0)] * 2 + pairs + [(0, 0, 0)] * 2
    meta = jnp.asarray(np.array(slots, dtype=np.int32).T)
    S = L // 2 + 1

    acc = pl.pallas_call(
        functools.partial(_loss_kernel, bm=BM, bn=BN),
        grid_spec=pltpu.PrefetchScalarGridSpec(
            num_scalar_prefetch=1,
            grid=(S,),
            in_specs=[
                pl.BlockSpec((D, B), lambda s, m: (0, 0)),
                pl.BlockSpec((BM, 2), lambda s, m: (m[0, 2 * s], 0)),
                pl.BlockSpec((BM, 2), lambda s, m: (m[0, 2 * s + 1], 0)),
                pl.BlockSpec((2, B), lambda s, m: (0, 0)),
            ],
            out_specs=pl.BlockSpec((8, BN), lambda s, m: (0, 0)),
            scratch_shapes=[
                pltpu.VMEM((BM, BN), jnp.float32),
                pltpu.VMEM((BM, BN), jnp.float32),
            ],
        ),
        out_shape=jax.ShapeDtypeStruct((8, BN), jnp.float32),
        compiler_params=pltpu.CompilerParams(
            dimension_semantics=("arbitrary",),
        ),
    )(meta, ebt, rowmeta, rowmeta, colmeta)

    total = jnp.sum(acc)
    return total / (2.0 * B * (B - 1))


# speedup vs baseline: 1.2591x; 1.2591x over previous
import functools

import jax
import jax.numpy as jnp
import numpy as np
from jax.experimental import pallas as pl
from jax.experimental.pallas import tpu as pltpu


def _prep_kernel(e_ref, eb_ref, u_ref, *, scale):
    e = e_ref[...] * scale
    eb_ref[...] = e.astype(jnp.bfloat16)
    u_ref[...] = 0.5 * jnp.sum(e * e, axis=1, keepdims=True)


def _loss_kernel(m_ref, eb_ref, rm0_ref, rm1_ref, cm_ref,
                 out_ref, ga_ref, gb_ref, gc_ref, gd_ref, *, bm, bn):
    s = pl.program_id(0)
    k0 = 2 * s

    @pl.when(s == 0)
    def _init():
        gc_ref[...] = jnp.zeros_like(gc_ref)
        gd_ref[...] = jnp.zeros_like(gd_ref)
        out_ref[...] = jnp.zeros_like(out_ref)

    def epilogue(g_ref, rm_ref, k):
        bj = m_ref[1, k]
        w = m_ref[2, k].astype(jnp.float32)
        col = pl.multiple_of(bj * bn, bn)
        li = rm_ref[:, 0:1]
        ui = rm_ref[:, 1:2]
        lj = cm_ref[0:1, pl.ds(col, bn)]
        uj = cm_ref[1:2, pl.ds(col, bn)]
        g = g_ref[...]
        mse = (ui + uj) - g
        val = jnp.where(li == lj, mse, jnp.maximum(1.0 - mse, 0.0))
        part = jnp.sum(val.reshape(bm // 8, 8, bn), axis=0)
        out_ref[...] += part * w

    def matmul(g_ref, k):
        bi = m_ref[0, k]
        bj = m_ref[1, k]
        row = pl.multiple_of(bi * bm, bm)
        col = pl.multiple_of(bj * bn, bn)
        ei = eb_ref[pl.ds(row, bm), :]
        ej = eb_ref[pl.ds(col, bn), :]
        g_ref[...] = jax.lax.dot_general(
            ei, ej, (((1,), (1,)), ((), ())),
            preferred_element_type=jnp.float32)

    even = jax.lax.rem(s, 2) == 0

    @pl.when(even)
    def _even_step():
        epilogue(gc_ref, rm0_ref, k0)
        epilogue(gd_ref, rm1_ref, k0 + 1)
        matmul(ga_ref, k0 + 2)
        matmul(gb_ref, k0 + 3)

    @pl.when(jnp.logical_not(even))
    def _odd_step():
        epilogue(ga_ref, rm0_ref, k0)
        epilogue(gb_ref, rm1_ref, k0 + 1)
        matmul(gc_ref, k0 + 2)
        matmul(gd_ref, k0 + 3)


def kernel(embeddings, labels):
    B, D = embeddings.shape
    BM = 512
    BN = 512
    nb = B // BM

    labf = labels.astype(jnp.float32)
    scale = 1.0 / float(np.sqrt(D / 2))

    NB = 1024
    eb, u_col = pl.pallas_call(
        functools.partial(_prep_kernel, scale=scale),
        grid=(B // NB,),
        in_specs=[pl.BlockSpec((NB, D), lambda i: (i, 0))],
        out_specs=[
            pl.BlockSpec((NB, D), lambda i: (i, 0)),
            pl.BlockSpec((NB, 1), lambda i: (i, 0)),
        ],
        out_shape=[
            jax.ShapeDtypeStruct((B, D), jnp.bfloat16),
            jax.ShapeDtypeStruct((B, 1), jnp.float32),
        ],
    )(embeddings)

    u = u_col[:, 0]
    colmeta = jnp.stack([labf, u], axis=0)
    rowmeta = jnp.stack([labf, u], axis=1)

    pairs = [(i, j, 1 if i == j else 2)
             for i in range(nb) for j in range(i, nb)]
    L = len(pairs)
    slots = [(0, 0, 0)] * 2 + pairs + [(0, 0, 0)] * 2
    meta = jnp.asarray(np.array(slots, dtype=np.int32).T)
    S = L // 2 + 1

    acc = pl.pallas_call(
        functools.partial(_loss_kernel, bm=BM, bn=BN),
        grid_spec=pltpu.PrefetchScalarGridSpec(
            num_scalar_prefetch=1,
            grid=(S,),
            in_specs=[
                pl.BlockSpec((B, D), lambda s, m: (0, 0)),
                pl.BlockSpec((BM, 2), lambda s, m: (m[0, 2 * s], 0)),
                pl.BlockSpec((BM, 2), lambda s, m: (m[0, 2 * s + 1], 0)),
                pl.BlockSpec((2, B), lambda s, m: (0, 0)),
            ],
            out_specs=pl.BlockSpec((8, BN), lambda s, m: (0, 0)),
            scratch_shapes=[
                pltpu.VMEM((BM, BN), jnp.float32),
                pltpu.VMEM((BM, BN), jnp.float32),
                pltpu.VMEM((BM, BN), jnp.float32),
                pltpu.VMEM((BM, BN), jnp.float32),
            ],
        ),
        out_shape=jax.ShapeDtypeStruct((8, BN), jnp.float32),
        compiler_params=pltpu.CompilerParams(
            dimension_semantics=("arbitrary",),
        ),
    )(meta, eb, rowmeta, rowmeta, colmeta)

    total = jnp.sum(acc)
    return total / (2.0 * B * (B - 1))
